# Initial kernel scaffold; baseline (speedup 1.0000x reference)
import functools

import jax
import jax.numpy as jnp
from jax import lax
from jax.experimental import pallas as pl
from jax.experimental.pallas import tpu as pltpu
from jax.experimental.pallas import tpu_sc as plsc

NC = 2
NS = 16
NW = NC * NS


def _pre_body(h_ref, wa_ref, wb_ref, be1_ref, hs_ref, hd_ref):
    h = h_ref[...]
    hs_ref[...] = jnp.dot(h, wa_ref[...], preferred_element_type=jnp.float32)
    hd_ref[...] = (
        jnp.dot(h, wb_ref[...], preferred_element_type=jnp.float32) + be1_ref[...]
    )


def _precompute(h, W_e1a, W_e1b, b_e1, bn=256):
    n, d = h.shape
    grid = (pl.cdiv(n, bn),)
    return pl.pallas_call(
        _pre_body,
        grid=grid,
        in_specs=[
            pl.BlockSpec((bn, d), lambda i: (i, 0)),
            pl.BlockSpec((d, d), lambda i: (0, 0)),
            pl.BlockSpec((d, d), lambda i: (0, 0)),
            pl.BlockSpec((1, d), lambda i: (0, 0)),
        ],
        out_specs=[
            pl.BlockSpec((bn, d), lambda i: (i, 0)),
            pl.BlockSpec((bn, d), lambda i: (i, 0)),
        ],
        out_shape=[
            jax.ShapeDtypeStruct((n, d), jnp.float32),
            jax.ShapeDtypeStruct((n, d), jnp.float32),
        ],
    )(h, W_e1a, W_e1b, b_e1)


def _make_gather(E, D, C):
    G = C // 128
    NCHUNK = E // C
    MAXJ = pl.cdiv(NCHUNK, NW)
    mesh = plsc.VectorSubcoreMesh(core_axis_name="c", subcore_axis_name="s")

    @functools.partial(
        pl.kernel,
        mesh=mesh,
        out_type=[
            jax.ShapeDtypeStruct((E, D), jnp.float32),
            jax.ShapeDtypeStruct((E, 16), jnp.float32),
        ],
        scratch_types=[
            pltpu.VMEM((G, 128), jnp.int32),
            pltpu.VMEM((G, 128), jnp.int32),
            pltpu.VMEM((C, D), jnp.float32),
            pltpu.VMEM((C, D), jnp.float32),
            pltpu.VMEM((C, 16), jnp.float32),
            pltpu.VMEM((C, 16), jnp.float32),
            pltpu.SemaphoreType.DMA,
        ],
    )
    def gather_kernel(
        hs_hbm, hd_hbm, crd_hbm, row2_hbm, col2_hbm,
        pre1_hbm, diff_hbm,
        idxr, idxc, bufA, bufB, bufR, bufC, sem,
    ):
        w = lax.axis_index("s") * NC + lax.axis_index("c")

        def chunk_body(j, carry):
            ch = w + j * NW

            @pl.when(ch < NCHUNK)
            def _():
                pltpu.sync_copy(row2_hbm.at[pl.ds(ch * G, G)], idxr)
                pltpu.sync_copy(col2_hbm.at[pl.ds(ch * G, G)], idxc)
                for g in range(G):
                    pltpu.async_copy(
                        hs_hbm.at[idxr.at[g]], bufA.at[pl.ds(g * 128, 128)], sem
                    ).wait()
                    pltpu.async_copy(
                        hd_hbm.at[idxc.at[g]], bufB.at[pl.ds(g * 128, 128)], sem
                    ).wait()
                    pltpu.async_copy(
                        crd_hbm.at[idxr.at[g]], bufR.at[pl.ds(g * 128, 128)], sem
                    ).wait()
                    pltpu.async_copy(
                        crd_hbm.at[idxc.at[g]], bufC.at[pl.ds(g * 128, 128)], sem
                    ).wait()

                def edge_body(e, c2):
                    for k in range(D // 16):
                        s = pl.ds(k * 16, 16)
                        bufA[e, s] = bufA[e, s] + bufB[e, s]
                    bufR[e, :] = bufR[e, :] - bufC[e, :]
                    return c2

                lax.fori_loop(0, C, edge_body, 0)
                pltpu.sync_copy(bufA, pre1_hbm.at[pl.ds(ch * C, C)])
                pltpu.sync_copy(bufR, diff_hbm.at[pl.ds(ch * C, C)])

            return carry

        lax.fori_loop(0, MAXJ, chunk_body, 0)

    return gather_kernel


def _edge_body(
    pre1_ref, diff_ref, wr_ref, we2_ref, be2_ref, watt_ref, batt_ref,
    wc1_ref, bc1_ref, wc2_ref, ef_ref, trans_ref,
):
    d = diff_ref[...]
    radial = jnp.sum(d * d, axis=1, keepdims=True)
    m1 = jax.nn.silu(pre1_ref[...] + radial * wr_ref[...])
    m2 = jax.nn.silu(
        jnp.dot(m1, we2_ref[...], preferred_element_type=jnp.float32) + be2_ref[...]
    )
    att_logit = jnp.sum(m2 * watt_ref[...], axis=1, keepdims=True) + batt_ref[0, 0]
    ef = m2 * jax.nn.sigmoid(att_logit)
    cm = jax.nn.silu(
        jnp.dot(ef, wc1_ref[...], preferred_element_type=jnp.float32) + bc1_ref[...]
    )
    c = jnp.sum(cm * wc2_ref[...], axis=1, keepdims=True)
    ef_ref[...] = ef
    trans_ref[...] = d * c


def _edge_mlp(pre1, diff, wr, W_e2, b_e2, watt, batt, W_c1, b_c1, wc2, be=512):
    E, D = pre1.shape
    grid = (E // be,)
    return pl.pallas_call(
        _edge_body,
        grid=grid,
        in_specs=[
            pl.BlockSpec((be, D), lambda i: (i, 0)),
            pl.BlockSpec((be, 16), lambda i: (i, 0)),
            pl.BlockSpec((1, D), lambda i: (0, 0)),
            pl.BlockSpec((D, D), lambda i: (0, 0)),
            pl.BlockSpec((1, D), lambda i: (0, 0)),
            pl.BlockSpec((1, D), lambda i: (0, 0)),
            pl.BlockSpec((1, 1), lambda i: (0, 0)),
            pl.BlockSpec((D, D), lambda i: (0, 0)),
            pl.BlockSpec((1, D), lambda i: (0, 0)),
            pl.BlockSpec((1, D), lambda i: (0, 0)),
        ],
        out_specs=[
            pl.BlockSpec((be, D), lambda i: (i, 0)),
            pl.BlockSpec((be, 16), lambda i: (i, 0)),
        ],
        out_shape=[
            jax.ShapeDtypeStruct((E, D), jnp.float32),
            jax.ShapeDtypeStruct((E, 16), jnp.float32),
        ],
    )(pre1, diff, wr, W_e2, b_e2, watt, batt, W_c1, b_c1, wc2)


def _make_scatter(E, N, D, C):
    G = C // 128
    NCHUNK = E // C
    MAXJ = pl.cdiv(NCHUNK, NW)
    RPT = N // NS
    mesh = plsc.VectorSubcoreMesh(core_axis_name="c", subcore_axis_name="s")

    @functools.partial(
        pl.kernel,
        mesh=mesh,
        out_type=[
            jax.ShapeDtypeStruct((NC, N, D), jnp.float32),
            jax.ShapeDtypeStruct((NC, N, 16), jnp.float32),
        ],
        scratch_types=[
            pltpu.VMEM((G, 128), jnp.int32),
            pltpu.VMEM((C, D), jnp.float32),
            pltpu.VMEM((C, 16), jnp.float32),
            pltpu.VMEM_SHARED((N, D), jnp.float32),
            pltpu.VMEM_SHARED((N, 16), jnp.float32),
        ],
    )
    def scatter_kernel(
        ef_hbm, trans_hbm, row2_hbm, z128_hbm, z16_hbm,
        agg_hbm, cagg_hbm,
        idx, featbuf, transbuf, acc_sh, cacc_sh,
    ):
        core = lax.axis_index("c")
        sid = lax.axis_index("s")
        w = sid * NC + core

        r0 = sid * RPT
        pltpu.sync_copy(z128_hbm.at[pl.ds(r0, RPT)], acc_sh.at[pl.ds(r0, RPT)])
        pltpu.sync_copy(z16_hbm.at[pl.ds(r0, RPT)], cacc_sh.at[pl.ds(r0, RPT)])
        plsc.subcore_barrier()

        def chunk_body(j, carry):
            ch = w + j * NW

            @pl.when(ch < NCHUNK)
            def _():
                pltpu.sync_copy(row2_hbm.at[pl.ds(ch * G, G)], idx)
                pltpu.sync_copy(ef_hbm.at[pl.ds(ch * C, C)], featbuf)
                pltpu.sync_copy(trans_hbm.at[pl.ds(ch * C, C)], transbuf)
                for g in range(G):
                    pltpu.sync_copy(
                        featbuf.at[pl.ds(g * 128, 128)],
                        acc_sh.at[idx.at[g]],
                        add=True,
                    )
                    pltpu.sync_copy(
                        transbuf.at[pl.ds(g * 128, 128)],
                        cacc_sh.at[idx.at[g]],
                        add=True,
                    )

            return carry

        lax.fori_loop(0, MAXJ, chunk_body, 0)
        plsc.subcore_barrier()

        pltpu.sync_copy(acc_sh.at[pl.ds(r0, RPT)], agg_hbm.at[core, pl.ds(r0, RPT)])
        pltpu.sync_copy(cacc_sh.at[pl.ds(r0, RPT)], cagg_hbm.at[core, pl.ds(r0, RPT)])

    return scatter_kernel


def _node_body(
    h_ref, agg_ref, cagg_ref, crd_ref, wna_ref, wnb_ref, bn1_ref, wn2_ref, bn2_ref,
    hout_ref, cout_ref,
):
    h = h_ref[...]
    agg = agg_ref[0] + agg_ref[1]
    x = jax.nn.silu(
        jnp.dot(h, wna_ref[...], preferred_element_type=jnp.float32)
        + jnp.dot(agg, wnb_ref[...], preferred_element_type=jnp.float32)
        + bn1_ref[...]
    )
    hout_ref[...] = (
        jnp.dot(x, wn2_ref[...], preferred_element_type=jnp.float32) + bn2_ref[...] + h
    )
    cout_ref[...] = crd_ref[...] + cagg_ref[0] + cagg_ref[1]


def _node_mlp(h, agg2, cagg2, crd16, W_n1a, W_n1b, b_n1, W_n2, b_n2, bn=256):
    n, d = h.shape
    grid = (pl.cdiv(n, bn),)
    return pl.pallas_call(
        _node_body,
        grid=grid,
        in_specs=[
            pl.BlockSpec((bn, d), lambda i: (i, 0)),
            pl.BlockSpec((NC, bn, d), lambda i: (0, i, 0)),
            pl.BlockSpec((NC, bn, 16), lambda i: (0, i, 0)),
            pl.BlockSpec((bn, 16), lambda i: (i, 0)),
            pl.BlockSpec((d, d), lambda i: (0, 0)),
            pl.BlockSpec((d, d), lambda i: (0, 0)),
            pl.BlockSpec((1, d), lambda i: (0, 0)),
            pl.BlockSpec((d, d), lambda i: (0, 0)),
            pl.BlockSpec((1, d), lambda i: (0, 0)),
        ],
        out_specs=[
            pl.BlockSpec((bn, d), lambda i: (i, 0)),
            pl.BlockSpec((bn, 16), lambda i: (i, 0)),
        ],
        out_shape=[
            jax.ShapeDtypeStruct((n, d), jnp.float32),
            jax.ShapeDtypeStruct((n, 16), jnp.float32),
        ],
    )(h, agg2, cagg2, crd16, W_n1a, W_n1b, b_n1, W_n2, b_n2)


def kernel(
    h, edge_index, coord,
    W_e1, b_e1, W_e2, b_e2, W_att, b_att,
    W_n1, b_n1, W_n2, b_n2, W_c1, b_c1, W_c2,
):
    N, D = h.shape
    E = edge_index.shape[1]
    assert E % (NW * 128) == 0 and D % 16 == 0 and N % NS == 0

    row2 = edge_index[0].reshape(E // 128, 128)
    col2 = edge_index[1].reshape(E // 128, 128)
    crd16 = jnp.pad(coord, ((0, 0), (0, 16 - coord.shape[1])))

    hs, hd = _precompute(h, W_e1[:D], W_e1[D : 2 * D], b_e1.reshape(1, D))

    pre1, diff = _make_gather(E, D, C=256)(hs, hd, crd16, row2, col2)

    wr = W_e1[2 * D].reshape(1, D)
    ef, trans = _edge_mlp(
        pre1, diff, wr, W_e2, b_e2.reshape(1, -1),
        W_att.reshape(1, D), b_att.reshape(1, 1),
        W_c1, b_c1.reshape(1, -1), W_c2.reshape(1, D),
    )

    z128 = jnp.zeros((N, D), jnp.float32)
    z16 = jnp.zeros((N, 16), jnp.float32)
    agg2, cagg2 = _make_scatter(E, N, D, C=512)(ef, trans, row2, z128, z16)

    h_out, c16 = _node_mlp(
        h, agg2, cagg2, crd16,
        W_n1[:D], W_n1[D:], b_n1.reshape(1, -1), W_n2, b_n2.reshape(1, -1),
    )
    return (h_out, c16[:, : coord.shape[1]])


# speedup vs baseline: 2.9161x; 2.9161x over previous
import functools

import jax
import jax.numpy as jnp
from jax import lax
from jax.experimental import pallas as pl
from jax.experimental.pallas import tpu as pltpu
from jax.experimental.pallas import tpu_sc as plsc

NC = 2
NS = 16
NW = NC * NS


def _pre_body(h_ref, wa_ref, wb_ref, be1_ref, hs_ref, hd_ref):
    h = h_ref[...]
    hs_ref[...] = jnp.dot(h, wa_ref[...], preferred_element_type=jnp.float32)
    hd_ref[...] = (
        jnp.dot(h, wb_ref[...], preferred_element_type=jnp.float32) + be1_ref[...]
    )


def _precompute(h, W_e1a, W_e1b, b_e1, bn=256):
    n, d = h.shape
    grid = (pl.cdiv(n, bn),)
    return pl.pallas_call(
        _pre_body,
        grid=grid,
        in_specs=[
            pl.BlockSpec((bn, d), lambda i: (i, 0)),
            pl.BlockSpec((d, d), lambda i: (0, 0)),
            pl.BlockSpec((d, d), lambda i: (0, 0)),
            pl.BlockSpec((1, d), lambda i: (0, 0)),
        ],
        out_specs=[
            pl.BlockSpec((bn, d), lambda i: (i, 0)),
            pl.BlockSpec((bn, d), lambda i: (i, 0)),
        ],
        out_shape=[
            jax.ShapeDtypeStruct((n, d), jnp.float32),
            jax.ShapeDtypeStruct((n, d), jnp.float32),
        ],
    )(h, W_e1a, W_e1b, b_e1)


def _dist_rows(src, dst, sid, n):
    ra = 8 * ((n + 8 * NS - 1) // (8 * NS))
    last = n - (NS - 1) * ra
    assert last > 0 and last % 8 == 0 and ra % 8 == 0

    @pl.when(sid < NS - 1)
    def _():
        pltpu.sync_copy(src.at[pl.ds(sid * ra, ra)], dst.at[pl.ds(sid * ra, ra)])

    @pl.when(sid == NS - 1)
    def _():
        pltpu.sync_copy(
            src.at[pl.ds((NS - 1) * ra, last)], dst.at[pl.ds((NS - 1) * ra, last)]
        )


def _make_gather(N, E, D, C):
    G = C // 128
    NCHUNK = E // C
    MAXJ = pl.cdiv(NCHUNK, NW)
    RPT = N // NS
    mesh = plsc.VectorSubcoreMesh(core_axis_name="c", subcore_axis_name="s")

    @functools.partial(
        pl.kernel,
        mesh=mesh,
        out_type=[
            jax.ShapeDtypeStruct((E, D), jnp.float32),
            jax.ShapeDtypeStruct((E, 16), jnp.float32),
        ],
        scratch_types=[
            pltpu.VMEM((G, 128), jnp.int32),
            pltpu.VMEM((G, 128), jnp.int32),
            pltpu.VMEM((C, D), jnp.float32),
            pltpu.VMEM((C, D), jnp.float32),
            pltpu.VMEM((C, 16), jnp.float32),
            pltpu.VMEM((C, 16), jnp.float32),
            pltpu.VMEM_SHARED((N, 16), jnp.float32),
            pltpu.SemaphoreType.DMA,
        ],
        compiler_params=pltpu.CompilerParams(use_tc_tiling_on_sc=False),
    )
    def gather_kernel(
        hs_hbm, hd_hbm, crd_hbm, row2_hbm, col2_hbm,
        pre1_hbm, diff_hbm,
        idxr, idxc, bufA, bufB, bufR, bufC, crd_sh, sem,
    ):
        core = lax.axis_index("c")
        sid = lax.axis_index("s")
        w = sid * NC + core

        _dist_rows(crd_hbm, crd_sh, sid, N)
        plsc.subcore_barrier()

        def chunk_body(j, carry):
            ch = w + j * NW

            @pl.when(ch < NCHUNK)
            def _():
                pltpu.sync_copy(row2_hbm.at[pl.ds(ch * G, G)], idxr)
                pltpu.sync_copy(col2_hbm.at[pl.ds(ch * G, G)], idxc)
                for g in range(G):
                    pltpu.async_copy(
                        hs_hbm.at[idxr.at[g]], bufA.at[pl.ds(g * 128, 128)], sem
                    ).wait()
                    pltpu.async_copy(
                        hd_hbm.at[idxc.at[g]], bufB.at[pl.ds(g * 128, 128)], sem
                    ).wait()
                    pltpu.async_copy(
                        crd_sh.at[idxr.at[g]], bufR.at[pl.ds(g * 128, 128)], sem
                    ).wait()
                    pltpu.async_copy(
                        crd_sh.at[idxc.at[g]], bufC.at[pl.ds(g * 128, 128)], sem
                    ).wait()

                def edge_body(e, c2):
                    for k in range(D // 16):
                        s = pl.ds(k * 16, 16)
                        bufA[e, s] = bufA[e, s] + bufB[e, s]
                    bufR[e, :] = bufR[e, :] - bufC[e, :]
                    return c2

                lax.fori_loop(0, C, edge_body, 0)
                pltpu.sync_copy(bufA, pre1_hbm.at[pl.ds(ch * C, C)])
                pltpu.sync_copy(bufR, diff_hbm.at[pl.ds(ch * C, C)])

            return carry

        lax.fori_loop(0, MAXJ, chunk_body, 0)

    return gather_kernel


def _edge_body(
    pre1_ref, diff_ref, wr_ref, we2_ref, be2_ref, watt_ref, batt_ref,
    wc1_ref, bc1_ref, wc2_ref, ef_ref, trans_ref,
):
    d = diff_ref[...]
    radial = jnp.sum(d * d, axis=1, keepdims=True)
    m1 = jax.nn.silu(pre1_ref[...] + radial * wr_ref[...])
    m2 = jax.nn.silu(
        jnp.dot(m1, we2_ref[...], preferred_element_type=jnp.float32) + be2_ref[...]
    )
    att_logit = jnp.sum(m2 * watt_ref[...], axis=1, keepdims=True) + batt_ref[0, 0]
    ef = m2 * jax.nn.sigmoid(att_logit)
    cm = jax.nn.silu(
        jnp.dot(ef, wc1_ref[...], preferred_element_type=jnp.float32) + bc1_ref[...]
    )
    c = jnp.sum(cm * wc2_ref[...], axis=1, keepdims=True)
    ef_ref[...] = ef
    trans_ref[...] = d * c


def _edge_mlp(pre1, diff, wr, W_e2, b_e2, watt, batt, W_c1, b_c1, wc2, be=512):
    E, D = pre1.shape
    grid = (E // be,)
    return pl.pallas_call(
        _edge_body,
        grid=grid,
        in_specs=[
            pl.BlockSpec((be, D), lambda i: (i, 0)),
            pl.BlockSpec((be, 16), lambda i: (i, 0)),
            pl.BlockSpec((1, D), lambda i: (0, 0)),
            pl.BlockSpec((D, D), lambda i: (0, 0)),
            pl.BlockSpec((1, D), lambda i: (0, 0)),
            pl.BlockSpec((1, D), lambda i: (0, 0)),
            pl.BlockSpec((1, 1), lambda i: (0, 0)),
            pl.BlockSpec((D, D), lambda i: (0, 0)),
            pl.BlockSpec((1, D), lambda i: (0, 0)),
            pl.BlockSpec((1, D), lambda i: (0, 0)),
        ],
        out_specs=[
            pl.BlockSpec((be, D), lambda i: (i, 0)),
            pl.BlockSpec((be, 16), lambda i: (i, 0)),
        ],
        out_shape=[
            jax.ShapeDtypeStruct((E, D), jnp.float32),
            jax.ShapeDtypeStruct((E, 16), jnp.float32),
        ],
    )(pre1, diff, wr, W_e2, b_e2, watt, batt, W_c1, b_c1, wc2)


def _make_scatter(E, N, D, C):
    G = C // 128
    NCHUNK = E // C
    MAXJ = pl.cdiv(NCHUNK, NW)
    RPT = N // NS
    mesh = plsc.VectorSubcoreMesh(core_axis_name="c", subcore_axis_name="s")

    @functools.partial(
        pl.kernel,
        mesh=mesh,
        out_type=[
            jax.ShapeDtypeStruct((NC, N, D), jnp.float32),
            jax.ShapeDtypeStruct((NC, N, 16), jnp.float32),
        ],
        scratch_types=[
            pltpu.VMEM((G, 128), jnp.int32),
            pltpu.VMEM((C, D), jnp.float32),
            pltpu.VMEM((C, 16), jnp.float32),
            pltpu.VMEM_SHARED((N, D), jnp.float32),
            pltpu.VMEM_SHARED((N, 16), jnp.float32),
        ],
        compiler_params=pltpu.CompilerParams(use_tc_tiling_on_sc=False),
    )
    def scatter_kernel(
        ef_hbm, trans_hbm, row2_hbm, z128_hbm, z16_hbm,
        agg_hbm, cagg_hbm,
        idx, featbuf, transbuf, acc_sh, cacc_sh,
    ):
        core = lax.axis_index("c")
        sid = lax.axis_index("s")
        w = sid * NC + core

        _dist_rows(z128_hbm, acc_sh, sid, N)
        _dist_rows(z16_hbm, cacc_sh, sid, N)
        plsc.subcore_barrier()

        def chunk_body(j, carry):
            ch = w + j * NW

            @pl.when(ch < NCHUNK)
            def _():
                pltpu.sync_copy(row2_hbm.at[pl.ds(ch * G, G)], idx)
                pltpu.sync_copy(ef_hbm.at[pl.ds(ch * C, C)], featbuf)
                pltpu.sync_copy(trans_hbm.at[pl.ds(ch * C, C)], transbuf)
                for g in range(G):
                    pltpu.sync_copy(
                        featbuf.at[pl.ds(g * 128, 128)],
                        acc_sh.at[idx.at[g]],
                        add=True,
                    )
                    pltpu.sync_copy(
                        transbuf.at[pl.ds(g * 128, 128)],
                        cacc_sh.at[idx.at[g]],
                        add=True,
                    )

            return carry

        lax.fori_loop(0, MAXJ, chunk_body, 0)
        plsc.subcore_barrier()

        _dist_rows(acc_sh, agg_hbm.at[core], sid, N)
        _dist_rows(cacc_sh, cagg_hbm.at[core], sid, N)

    return scatter_kernel


def _node_body(
    h_ref, agg_ref, cagg_ref, crd_ref, wna_ref, wnb_ref, bn1_ref, wn2_ref, bn2_ref,
    hout_ref, cout_ref,
):
    h = h_ref[...]
    agg = agg_ref[0] + agg_ref[1]
    x = jax.nn.silu(
        jnp.dot(h, wna_ref[...], preferred_element_type=jnp.float32)
        + jnp.dot(agg, wnb_ref[...], preferred_element_type=jnp.float32)
        + bn1_ref[...]
    )
    hout_ref[...] = (
        jnp.dot(x, wn2_ref[...], preferred_element_type=jnp.float32) + bn2_ref[...] + h
    )
    cout_ref[...] = crd_ref[...] + cagg_ref[0] + cagg_ref[1]


def _node_mlp(h, agg2, cagg2, crd16, W_n1a, W_n1b, b_n1, W_n2, b_n2, bn=256):
    n, d = h.shape
    grid = (pl.cdiv(n, bn),)
    return pl.pallas_call(
        _node_body,
        grid=grid,
        in_specs=[
            pl.BlockSpec((bn, d), lambda i: (i, 0)),
            pl.BlockSpec((NC, bn, d), lambda i: (0, i, 0)),
            pl.BlockSpec((NC, bn, 16), lambda i: (0, i, 0)),
            pl.BlockSpec((bn, 16), lambda i: (i, 0)),
            pl.BlockSpec((d, d), lambda i: (0, 0)),
            pl.BlockSpec((d, d), lambda i: (0, 0)),
            pl.BlockSpec((1, d), lambda i: (0, 0)),
            pl.BlockSpec((d, d), lambda i: (0, 0)),
            pl.BlockSpec((1, d), lambda i: (0, 0)),
        ],
        out_specs=[
            pl.BlockSpec((bn, d), lambda i: (i, 0)),
            pl.BlockSpec((bn, 16), lambda i: (i, 0)),
        ],
        out_shape=[
            jax.ShapeDtypeStruct((n, d), jnp.float32),
            jax.ShapeDtypeStruct((n, 16), jnp.float32),
        ],
    )(h, agg2, cagg2, crd16, W_n1a, W_n1b, b_n1, W_n2, b_n2)


def kernel(
    h, edge_index, coord,
    W_e1, b_e1, W_e2, b_e2, W_att, b_att,
    W_n1, b_n1, W_n2, b_n2, W_c1, b_c1, W_c2,
):
    N, D = h.shape
    E = edge_index.shape[1]
    assert E % 512 == 0 and D % 16 == 0 and N % NS == 0

    row2 = edge_index[0].reshape(E // 128, 128)
    col2 = edge_index[1].reshape(E // 128, 128)
    crd16 = jnp.pad(coord, ((0, 0), (0, 16 - coord.shape[1])))

    hs, hd = _precompute(h, W_e1[:D], W_e1[D : 2 * D], b_e1.reshape(1, D))

    pre1, diff = _make_gather(N, E, D, C=256)(hs, hd, crd16, row2, col2)

    wr = W_e1[2 * D].reshape(1, D)
    ef, trans = _edge_mlp(
        pre1, diff, wr, W_e2, b_e2.reshape(1, -1),
        W_att.reshape(1, D), b_att.reshape(1, 1),
        W_c1, b_c1.reshape(1, -1), W_c2.reshape(1, D),
    )

    z128 = jnp.zeros((N, D), jnp.float32)
    z16 = jnp.zeros((N, 16), jnp.float32)
    agg2, cagg2 = _make_scatter(E, N, D, C=256)(ef, trans, row2, z128, z16)

    h_out, c16 = _node_mlp(
        h, agg2, cagg2, crd16,
        W_n1[:D], W_n1[D:], b_n1.reshape(1, -1), W_n2, b_n2.reshape(1, -1),
    )
    return (h_out, c16[:, : coord.shape[1]])


# speedup vs baseline: 3.5882x; 1.2305x over previous
import functools

import jax
import jax.numpy as jnp
from jax import lax
from jax.experimental import pallas as pl
from jax.experimental.pallas import tpu as pltpu
from jax.experimental.pallas import tpu_sc as plsc

NC = 2
NS = 16
NW = NC * NS


def _pre_body(h_ref, wa_ref, wb_ref, be1_ref, hs_ref, hd_ref):
    h = h_ref[...]
    hs_ref[...] = jnp.dot(h, wa_ref[...], preferred_element_type=jnp.float32)
    hd_ref[...] = (
        jnp.dot(h, wb_ref[...], preferred_element_type=jnp.float32) + be1_ref[...]
    )


def _precompute(h, W_e1a, W_e1b, b_e1, bn=256):
    n, d = h.shape
    grid = (pl.cdiv(n, bn),)
    return pl.pallas_call(
        _pre_body,
        grid=grid,
        in_specs=[
            pl.BlockSpec((bn, d), lambda i: (i, 0)),
            pl.BlockSpec((d, d), lambda i: (0, 0)),
            pl.BlockSpec((d, d), lambda i: (0, 0)),
            pl.BlockSpec((1, d), lambda i: (0, 0)),
        ],
        out_specs=[
            pl.BlockSpec((bn, d), lambda i: (i, 0)),
            pl.BlockSpec((bn, d), lambda i: (i, 0)),
        ],
        out_shape=[
            jax.ShapeDtypeStruct((n, d), jnp.float32),
            jax.ShapeDtypeStruct((n, d), jnp.float32),
        ],
    )(h, W_e1a, W_e1b, b_e1)


def _dist_rows(src, dst, sid, n):
    ra = 8 * ((n + 8 * NS - 1) // (8 * NS))
    last = n - (NS - 1) * ra
    assert last > 0 and last % 8 == 0 and ra % 8 == 0

    @pl.when(sid < NS - 1)
    def _():
        pltpu.sync_copy(src.at[pl.ds(sid * ra, ra)], dst.at[pl.ds(sid * ra, ra)])

    @pl.when(sid == NS - 1)
    def _():
        pltpu.sync_copy(
            src.at[pl.ds((NS - 1) * ra, last)], dst.at[pl.ds((NS - 1) * ra, last)]
        )


def _make_gather(N, E, D):
    C = 64
    NCHUNK = E // C
    J = pl.cdiv(NCHUNK, NW)
    NB = 3
    TT = -(-J // NB)
    mesh = plsc.VectorSubcoreMesh(core_axis_name="c", subcore_axis_name="s")

    @functools.partial(
        pl.kernel,
        mesh=mesh,
        out_type=[
            jax.ShapeDtypeStruct((E, D), jnp.float32),
            jax.ShapeDtypeStruct((E, 16), jnp.float32),
        ],
        scratch_types=(
            [pltpu.VMEM((2, C), jnp.int32) for _ in range(NB)]
            + [pltpu.VMEM((C, D), jnp.float32) for _ in range(NB)]
            + [pltpu.VMEM((C, D), jnp.float32) for _ in range(NB)]
            + [pltpu.VMEM((C, 16), jnp.float32) for _ in range(NB)]
            + [pltpu.VMEM((C, 16), jnp.float32) for _ in range(NB)]
            + [pltpu.VMEM_SHARED((N, 16), jnp.float32)]
            + [pltpu.SemaphoreType.DMA] * (7 * NB)
        ),
        compiler_params=pltpu.CompilerParams(use_tc_tiling_on_sc=False),
    )
    def gather_kernel(hs_hbm, hd_hbm, crd_hbm, rc2_hbm, pre1_hbm, diff_hbm, *sc):
        idx = sc[0:NB]
        A = sc[NB : 2 * NB]
        B = sc[2 * NB : 3 * NB]
        R = sc[3 * NB : 4 * NB]
        Cc = sc[4 * NB : 5 * NB]
        crd_sh = sc[5 * NB]
        sems = sc[5 * NB + 1 :]
        semi = sems[0:NB]
        semg = [sems[NB + 4 * b : NB + 4 * b + 4] for b in range(NB)]
        semo = [sems[5 * NB + 2 * b : 5 * NB + 2 * b + 2] for b in range(NB)]

        core = lax.axis_index("c")
        sid = lax.axis_index("s")
        w = sid * NC + core

        _dist_rows(crd_hbm, crd_sh, sid, N)
        plsc.subcore_barrier()

        def cid(k):
            return jnp.minimum(w + k * NW, NCHUNK - 1)

        def idx_start(b, ch):
            pltpu.async_copy(rc2_hbm.at[ch], idx[b], semi[b])

        def idx_wait(b):
            pltpu.make_async_copy(rc2_hbm.at[0], idx[b], semi[b]).wait()

        def gather_start(b):
            pltpu.async_copy(hs_hbm.at[idx[b].at[0]], A[b], semg[b][0])
            pltpu.async_copy(hd_hbm.at[idx[b].at[1]], B[b], semg[b][1])
            pltpu.async_copy(crd_sh.at[idx[b].at[0]], R[b], semg[b][2])
            pltpu.async_copy(crd_sh.at[idx[b].at[1]], Cc[b], semg[b][3])

        def gather_wait(b):
            pltpu.make_async_copy(hs_hbm.at[idx[b].at[0]], A[b], semg[b][0]).wait()
            pltpu.make_async_copy(hd_hbm.at[idx[b].at[1]], B[b], semg[b][1]).wait()
            pltpu.make_async_copy(crd_hbm.at[pl.ds(0, C)], R[b], semg[b][2]).wait()
            pltpu.make_async_copy(crd_hbm.at[pl.ds(0, C)], Cc[b], semg[b][3]).wait()

        def compute(b):
            def edge_body(e, carry):
                for k in range(D // 16):
                    s = pl.ds(k * 16, 16)
                    A[b][e, s] = A[b][e, s] + B[b][e, s]
                R[b][e, :] = R[b][e, :] - Cc[b][e, :]
                return carry

            lax.fori_loop(0, C, edge_body, 0)

        def out_start(b, ch):
            pltpu.async_copy(A[b], pre1_hbm.at[pl.ds(ch * C, C)], semo[b][0])
            pltpu.async_copy(R[b], diff_hbm.at[pl.ds(ch * C, C)], semo[b][1])

        def out_wait(b):
            pltpu.make_async_copy(A[b], pre1_hbm.at[pl.ds(0, C)], semo[b][0]).wait()
            pltpu.make_async_copy(R[b], diff_hbm.at[pl.ds(0, C)], semo[b][1]).wait()

        def step(j, b, first):
            bn = (b + 1) % NB
            if not (first and b < NB - 1):
                out_wait(bn)
            idx_wait(bn)
            gather_start(bn)
            gather_wait(b)
            compute(b)
            out_start(b, cid(j * NB + b))
            idx_start(b, cid(j * NB + b + NB))

        for b in range(NB):
            idx_start(b, cid(b))
        idx_wait(0)
        gather_start(0)

        for b in range(NB):
            step(0, b, True)

        def loop_body(j, carry):
            for b in range(NB):
                step(j, b, False)
            return carry

        lax.fori_loop(1, TT, loop_body, 0)

        out_wait(1)
        out_wait(2)
        gather_wait(0)
        idx_wait(1)
        idx_wait(2)

    return gather_kernel


def _edge_body(
    pre1_ref, diff_ref, wr_ref, we2_ref, be2_ref, watt_ref, batt_ref,
    wc1_ref, bc1_ref, wc2_ref, ef_ref, trans_ref,
):
    d = diff_ref[...]
    radial = jnp.sum(d * d, axis=1, keepdims=True)
    m1 = jax.nn.silu(pre1_ref[...] + radial * wr_ref[...])
    m2 = jax.nn.silu(
        jnp.dot(m1, we2_ref[...], preferred_element_type=jnp.float32) + be2_ref[...]
    )
    att_logit = jnp.sum(m2 * watt_ref[...], axis=1, keepdims=True) + batt_ref[0, 0]
    ef = m2 * jax.nn.sigmoid(att_logit)
    cm = jax.nn.silu(
        jnp.dot(ef, wc1_ref[...], preferred_element_type=jnp.float32) + bc1_ref[...]
    )
    c = jnp.sum(cm * wc2_ref[...], axis=1, keepdims=True)
    ef_ref[...] = ef
    trans_ref[...] = (d * c)[:, :8]


def _edge_mlp(pre1, diff, wr, W_e2, b_e2, watt, batt, W_c1, b_c1, wc2, be=512):
    E, D = pre1.shape
    grid = (E // be,)
    return pl.pallas_call(
        _edge_body,
        grid=grid,
        in_specs=[
            pl.BlockSpec((be, D), lambda i: (i, 0)),
            pl.BlockSpec((be, 16), lambda i: (i, 0)),
            pl.BlockSpec((1, D), lambda i: (0, 0)),
            pl.BlockSpec((D, D), lambda i: (0, 0)),
            pl.BlockSpec((1, D), lambda i: (0, 0)),
            pl.BlockSpec((1, D), lambda i: (0, 0)),
            pl.BlockSpec((1, 1), lambda i: (0, 0)),
            pl.BlockSpec((D, D), lambda i: (0, 0)),
            pl.BlockSpec((1, D), lambda i: (0, 0)),
            pl.BlockSpec((1, D), lambda i: (0, 0)),
        ],
        out_specs=[
            pl.BlockSpec((be, D), lambda i: (i, 0)),
            pl.BlockSpec((be, 8), lambda i: (i, 0)),
        ],
        out_shape=[
            jax.ShapeDtypeStruct((E, D), jnp.float32),
            jax.ShapeDtypeStruct((E, 8), jnp.float32),
        ],
    )(pre1, diff, wr, W_e2, b_e2, watt, batt, W_c1, b_c1, wc2)


def _make_scatter(E, N, D):
    C = 64
    NCHUNK = E // C
    J = pl.cdiv(NCHUNK, NW)
    NB = 2
    TT = -(-J // NB)
    mesh = plsc.VectorSubcoreMesh(core_axis_name="c", subcore_axis_name="s")

    @functools.partial(
        pl.kernel,
        mesh=mesh,
        out_type=[
            jax.ShapeDtypeStruct((NC, N, D), jnp.float32),
            jax.ShapeDtypeStruct((NC, N, 8), jnp.float32),
        ],
        scratch_types=(
            [pltpu.VMEM((2, C), jnp.int32) for _ in range(NB)]
            + [pltpu.VMEM((C, D), jnp.float32) for _ in range(NB)]
            + [pltpu.VMEM((C, 8), jnp.float32) for _ in range(NB)]
            + [pltpu.VMEM_SHARED((N, D), jnp.float32)]
            + [pltpu.VMEM_SHARED((N, 8), jnp.float32)]
            + [pltpu.SemaphoreType.DMA] * (5 * NB)
        ),
        compiler_params=pltpu.CompilerParams(use_tc_tiling_on_sc=False),
    )
    def scatter_kernel(ef_hbm, trans_hbm, rc2_hbm, z8_hbm,
                       agg_hbm, cagg_hbm, *sc):
        idx = sc[0:NB]
        F = sc[NB : 2 * NB]
        T = sc[2 * NB : 3 * NB]
        acc_sh = sc[3 * NB]
        cacc_sh = sc[3 * NB + 1]
        sems = sc[3 * NB + 2 :]
        semr = [sems[3 * b : 3 * b + 3] for b in range(NB)]
        sema = [sems[3 * NB + 2 * b : 3 * NB + 2 * b + 2] for b in range(NB)]

        core = lax.axis_index("c")
        sid = lax.axis_index("s")
        w = sid * NC + core

        def zf(i, carry):
            for k in range(D // 16):
                F[0][i, pl.ds(k * 16, 16)] = jnp.zeros((16,), jnp.float32)
            return carry

        lax.fori_loop(0, C, zf, 0)
        ra = 8 * ((N + 8 * NS - 1) // (8 * NS))
        base = sid * ra
        last = N - (NS - 1) * ra

        def row_chunks(total):
            return [C] * (total // C) + ([total % C] if total % C else [])

        for tail, sizes in ((False, row_chunks(ra)), (True, row_chunks(last))):
            @pl.when((sid == NS - 1) if tail else (sid < NS - 1))
            def _():
                off = 0
                for sz in sizes:
                    pltpu.sync_copy(F[0].at[pl.ds(0, sz)],
                                    acc_sh.at[pl.ds(base + off, sz)])
                    off += sz

        _dist_rows(z8_hbm, cacc_sh, sid, N)
        plsc.subcore_barrier()

        def cid(k):
            return jnp.minimum(w + k * NW, NCHUNK - 1)

        def real(k):
            return w + k * NW < NCHUNK

        def read_start(b, ch):
            pltpu.async_copy(rc2_hbm.at[ch], idx[b], semr[b][0])
            pltpu.async_copy(ef_hbm.at[pl.ds(ch * C, C)], F[b], semr[b][1])
            pltpu.async_copy(trans_hbm.at[pl.ds(ch * C, C)], T[b], semr[b][2])

        def read_wait(b):
            pltpu.make_async_copy(rc2_hbm.at[0], idx[b], semr[b][0]).wait()
            pltpu.make_async_copy(ef_hbm.at[pl.ds(0, C)], F[b], semr[b][1]).wait()
            pltpu.make_async_copy(trans_hbm.at[pl.ds(0, C)], T[b], semr[b][2]).wait()

        def adds_start(b):
            pltpu.async_copy(F[b], acc_sh.at[idx[b].at[0]], sema[b][0], add=True)
            pltpu.async_copy(T[b], cacc_sh.at[idx[b].at[0]], sema[b][1], add=True)

        def adds_wait(b):
            pltpu.make_async_copy(F[b], acc_sh.at[idx[b].at[0]], sema[b][0]).wait()
            pltpu.make_async_copy(T[b], cacc_sh.at[idx[b].at[0]], sema[b][1]).wait()

        for b in range(NB):
            read_start(b, cid(b))

        def loop_body(j, carry):
            for b in range(NB):
                read_wait(b)

                @pl.when(real(j * NB + b))
                def _():
                    adds_start(b)

            for b in range(NB):

                @pl.when(real(j * NB + b))
                def _():
                    adds_wait(b)

                read_start(b, cid(j * NB + b + NB))
            return carry

        lax.fori_loop(0, TT, loop_body, 0)
        for b in range(NB):
            read_wait(b)
        plsc.subcore_barrier()

        for tail, sizes in ((False, row_chunks(ra)), (True, row_chunks(last))):
            @pl.when((sid == NS - 1) if tail else (sid < NS - 1))
            def _():
                off = 0
                for sz in sizes:
                    pltpu.sync_copy(acc_sh.at[pl.ds(base + off, sz)],
                                    F[0].at[pl.ds(0, sz)])
                    pltpu.sync_copy(F[0].at[pl.ds(0, sz)],
                                    agg_hbm.at[core, pl.ds(base + off, sz)])
                    pltpu.sync_copy(cacc_sh.at[pl.ds(base + off, sz)],
                                    T[0].at[pl.ds(0, sz)])
                    pltpu.sync_copy(T[0].at[pl.ds(0, sz)],
                                    cagg_hbm.at[core, pl.ds(base + off, sz)])
                    off += sz

    return scatter_kernel


def _node_body(
    h_ref, agg_ref, cagg_ref, crd_ref, wna_ref, wnb_ref, bn1_ref, wn2_ref, bn2_ref,
    hout_ref, cout_ref,
):
    h = h_ref[...]
    agg = agg_ref[0] + agg_ref[1]
    x = jax.nn.silu(
        jnp.dot(h, wna_ref[...], preferred_element_type=jnp.float32)
        + jnp.dot(agg, wnb_ref[...], preferred_element_type=jnp.float32)
        + bn1_ref[...]
    )
    hout_ref[...] = (
        jnp.dot(x, wn2_ref[...], preferred_element_type=jnp.float32) + bn2_ref[...] + h
    )
    cout_ref[...] = crd_ref[...] + cagg_ref[0] + cagg_ref[1]


def _node_mlp(h, agg2, cagg2, crd8, W_n1a, W_n1b, b_n1, W_n2, b_n2, bn=256):
    n, d = h.shape
    grid = (pl.cdiv(n, bn),)
    return pl.pallas_call(
        _node_body,
        grid=grid,
        in_specs=[
            pl.BlockSpec((bn, d), lambda i: (i, 0)),
            pl.BlockSpec((NC, bn, d), lambda i: (0, i, 0)),
            pl.BlockSpec((NC, bn, 8), lambda i: (0, i, 0)),
            pl.BlockSpec((bn, 8), lambda i: (i, 0)),
            pl.BlockSpec((d, d), lambda i: (0, 0)),
            pl.BlockSpec((d, d), lambda i: (0, 0)),
            pl.BlockSpec((1, d), lambda i: (0, 0)),
            pl.BlockSpec((d, d), lambda i: (0, 0)),
            pl.BlockSpec((1, d), lambda i: (0, 0)),
        ],
        out_specs=[
            pl.BlockSpec((bn, d), lambda i: (i, 0)),
            pl.BlockSpec((bn, 8), lambda i: (i, 0)),
        ],
        out_shape=[
            jax.ShapeDtypeStruct((n, d), jnp.float32),
            jax.ShapeDtypeStruct((n, 8), jnp.float32),
        ],
    )(h, agg2, cagg2, crd8, W_n1a, W_n1b, b_n1, W_n2, b_n2)


def kernel(
    h, edge_index, coord,
    W_e1, b_e1, W_e2, b_e2, W_att, b_att,
    W_n1, b_n1, W_n2, b_n2, W_c1, b_c1, W_c2,
):
    N, D = h.shape
    E = edge_index.shape[1]
    assert E % 512 == 0 and D % 16 == 0 and N % NS == 0

    rc64 = edge_index.reshape(2, E // 64, 64).transpose(1, 0, 2)
    crd16 = jnp.pad(coord, ((0, 0), (0, 16 - coord.shape[1])))

    hs, hd = _precompute(h, W_e1[:D], W_e1[D : 2 * D], b_e1.reshape(1, D))

    pre1, diff = _make_gather(N, E, D)(hs, hd, crd16, rc64)

    wr = W_e1[2 * D].reshape(1, D)
    ef, trans = _edge_mlp(
        pre1, diff, wr, W_e2, b_e2.reshape(1, -1),
        W_att.reshape(1, D), b_att.reshape(1, 1),
        W_c1, b_c1.reshape(1, -1), W_c2.reshape(1, D),
    )

    z8 = jnp.zeros((N, 8), jnp.float32)
    agg2, cagg2 = _make_scatter(E, N, D)(ef, trans, rc64, z8)

    h_out, c8 = _node_mlp(
        h, agg2, cagg2, crd16[:, :8],
        W_n1[:D], W_n1[D:], b_n1.reshape(1, -1), W_n2, b_n2.reshape(1, -1),
    )
    return (h_out, c8[:, : coord.shape[1]])


# speedup vs baseline: 4.0347x; 1.1244x over previous
import functools

import jax
import jax.numpy as jnp
from jax import lax
from jax.experimental import pallas as pl
from jax.experimental.pallas import tpu as pltpu
from jax.experimental.pallas import tpu_sc as plsc

NC = 2
NS = 16
NW = NC * NS


def _pre_body(h_ref, wa_ref, wb_ref, be1_ref, hs_ref, hd_ref):
    h = h_ref[...]
    hs_ref[...] = jnp.dot(h, wa_ref[...], preferred_element_type=jnp.float32)
    hd_ref[...] = (
        jnp.dot(h, wb_ref[...], preferred_element_type=jnp.float32) + be1_ref[...]
    )


def _precompute(h, W_e1a, W_e1b, b_e1, bn=256):
    n, d = h.shape
    grid = (pl.cdiv(n, bn),)
    return pl.pallas_call(
        _pre_body,
        grid=grid,
        in_specs=[
            pl.BlockSpec((bn, d), lambda i: (i, 0)),
            pl.BlockSpec((d, d), lambda i: (0, 0)),
            pl.BlockSpec((d, d), lambda i: (0, 0)),
            pl.BlockSpec((1, d), lambda i: (0, 0)),
        ],
        out_specs=[
            pl.BlockSpec((bn, d), lambda i: (i, 0)),
            pl.BlockSpec((bn, d), lambda i: (i, 0)),
        ],
        out_shape=[
            jax.ShapeDtypeStruct((n, d), jnp.float32),
            jax.ShapeDtypeStruct((n, d), jnp.float32),
        ],
    )(h, W_e1a, W_e1b, b_e1)


def _dist_rows(src, dst, sid, n):
    ra = 8 * ((n + 8 * NS - 1) // (8 * NS))
    last = n - (NS - 1) * ra
    assert last > 0 and last % 8 == 0 and ra % 8 == 0

    @pl.when(sid < NS - 1)
    def _():
        pltpu.sync_copy(src.at[pl.ds(sid * ra, ra)], dst.at[pl.ds(sid * ra, ra)])

    @pl.when(sid == NS - 1)
    def _():
        pltpu.sync_copy(
            src.at[pl.ds((NS - 1) * ra, last)], dst.at[pl.ds((NS - 1) * ra, last)]
        )


def _make_gather(N, E, D):
    C = 64
    NCHUNK = E // C
    J = pl.cdiv(NCHUNK, NW)
    NB = 3
    TT = -(-J // NB)
    mesh = plsc.VectorSubcoreMesh(core_axis_name="c", subcore_axis_name="s")

    @functools.partial(
        pl.kernel,
        mesh=mesh,
        out_type=[
            jax.ShapeDtypeStruct((E, D), jnp.float32),
            jax.ShapeDtypeStruct((E, 16), jnp.float32),
        ],
        scratch_types=(
            [pltpu.VMEM((2, C), jnp.int32) for _ in range(NB)]
            + [pltpu.VMEM((C, D), jnp.float32) for _ in range(NB)]
            + [pltpu.VMEM((C, D), jnp.float32) for _ in range(NB)]
            + [pltpu.VMEM((C, 16), jnp.float32) for _ in range(NB)]
            + [pltpu.VMEM((C, 16), jnp.float32) for _ in range(NB)]
            + [pltpu.VMEM_SHARED((N, 16), jnp.float32)]
            + [pltpu.SemaphoreType.DMA] * (7 * NB)
        ),
        compiler_params=pltpu.CompilerParams(use_tc_tiling_on_sc=False),
    )
    def gather_kernel(hs_hbm, hd_hbm, crd_hbm, rc2_hbm, pre1_hbm, diff_hbm, *sc):
        idx = sc[0:NB]
        A = sc[NB : 2 * NB]
        B = sc[2 * NB : 3 * NB]
        R = sc[3 * NB : 4 * NB]
        Cc = sc[4 * NB : 5 * NB]
        crd_sh = sc[5 * NB]
        sems = sc[5 * NB + 1 :]
        semi = sems[0:NB]
        semg = [sems[NB + 4 * b : NB + 4 * b + 4] for b in range(NB)]
        semo = [sems[5 * NB + 2 * b : 5 * NB + 2 * b + 2] for b in range(NB)]

        core = lax.axis_index("c")
        sid = lax.axis_index("s")
        w = sid * NC + core

        _dist_rows(crd_hbm, crd_sh, sid, N)
        plsc.subcore_barrier()

        def cid(k):
            return jnp.minimum(w + k * NW, NCHUNK - 1)

        def idx_start(b, ch):
            pltpu.async_copy(rc2_hbm.at[ch], idx[b], semi[b])

        def idx_wait(b):
            pltpu.make_async_copy(rc2_hbm.at[0], idx[b], semi[b]).wait()

        def gather_start(b):
            pltpu.async_copy(hs_hbm.at[idx[b].at[0]], A[b], semg[b][0])
            pltpu.async_copy(hd_hbm.at[idx[b].at[1]], B[b], semg[b][1])
            pltpu.async_copy(crd_sh.at[idx[b].at[0]], R[b], semg[b][2])
            pltpu.async_copy(crd_sh.at[idx[b].at[1]], Cc[b], semg[b][3])

        def gather_wait(b):
            pltpu.make_async_copy(hs_hbm.at[idx[b].at[0]], A[b], semg[b][0]).wait()
            pltpu.make_async_copy(hd_hbm.at[idx[b].at[1]], B[b], semg[b][1]).wait()
            pltpu.make_async_copy(crd_hbm.at[pl.ds(0, C)], R[b], semg[b][2]).wait()
            pltpu.make_async_copy(crd_hbm.at[pl.ds(0, C)], Cc[b], semg[b][3]).wait()

        def compute(b):
            def edge_body(e, carry):
                for k in range(D // 16):
                    s = pl.ds(k * 16, 16)
                    A[b][e, s] = A[b][e, s] + B[b][e, s]
                R[b][e, :] = R[b][e, :] - Cc[b][e, :]
                return carry

            lax.fori_loop(0, C, edge_body, 0)

        def out_start(b, ch):
            pltpu.async_copy(A[b], pre1_hbm.at[pl.ds(ch * C, C)], semo[b][0])
            pltpu.async_copy(R[b], diff_hbm.at[pl.ds(ch * C, C)], semo[b][1])

        def out_wait(b):
            pltpu.make_async_copy(A[b], pre1_hbm.at[pl.ds(0, C)], semo[b][0]).wait()
            pltpu.make_async_copy(R[b], diff_hbm.at[pl.ds(0, C)], semo[b][1]).wait()

        def step(j, b, first):
            bn = (b + 1) % NB
            if not (first and b < NB - 1):
                out_wait(bn)
            idx_wait(bn)
            gather_start(bn)
            gather_wait(b)
            compute(b)
            out_start(b, cid(j * NB + b))
            idx_start(b, cid(j * NB + b + NB))

        for b in range(NB):
            idx_start(b, cid(b))
        idx_wait(0)
        gather_start(0)

        for b in range(NB):
            step(0, b, True)

        def loop_body(j, carry):
            for b in range(NB):
                step(j, b, False)
            return carry

        lax.fori_loop(1, TT, loop_body, 0)

        out_wait(1)
        out_wait(2)
        gather_wait(0)
        idx_wait(1)
        idx_wait(2)

    return gather_kernel


def _edge_body(
    pre1_ref, diff_ref, wr_ref, we2_ref, be2_ref, watt_ref, batt_ref,
    wc1_ref, bc1_ref, wc2_ref, ef_ref, trans_ref,
):
    d = diff_ref[...]
    radial = jnp.sum(d * d, axis=1, keepdims=True)
    m1 = jax.nn.silu(pre1_ref[...] + radial * wr_ref[...])
    m2 = jax.nn.silu(
        jnp.dot(m1, we2_ref[...], preferred_element_type=jnp.float32) + be2_ref[...]
    )
    att_logit = jnp.sum(m2 * watt_ref[...], axis=1, keepdims=True) + batt_ref[0, 0]
    ef = m2 * jax.nn.sigmoid(att_logit)
    cm = jax.nn.silu(
        jnp.dot(ef, wc1_ref[...], preferred_element_type=jnp.float32) + bc1_ref[...]
    )
    c = jnp.sum(cm * wc2_ref[...], axis=1, keepdims=True)
    ef_ref[...] = ef
    trans_ref[...] = (d * c)[:, :8]


def _edge_mlp(pre1, diff, wr, W_e2, b_e2, watt, batt, W_c1, b_c1, wc2, be=512):
    E, D = pre1.shape
    grid = (E // be,)
    return pl.pallas_call(
        _edge_body,
        grid=grid,
        in_specs=[
            pl.BlockSpec((be, D), lambda i: (i, 0)),
            pl.BlockSpec((be, 16), lambda i: (i, 0)),
            pl.BlockSpec((1, D), lambda i: (0, 0)),
            pl.BlockSpec((D, D), lambda i: (0, 0)),
            pl.BlockSpec((1, D), lambda i: (0, 0)),
            pl.BlockSpec((1, D), lambda i: (0, 0)),
            pl.BlockSpec((1, 1), lambda i: (0, 0)),
            pl.BlockSpec((D, D), lambda i: (0, 0)),
            pl.BlockSpec((1, D), lambda i: (0, 0)),
            pl.BlockSpec((1, D), lambda i: (0, 0)),
        ],
        out_specs=[
            pl.BlockSpec((be, D), lambda i: (i, 0)),
            pl.BlockSpec((be, 8), lambda i: (i, 0)),
        ],
        out_shape=[
            jax.ShapeDtypeStruct((E, D), jnp.float32),
            jax.ShapeDtypeStruct((E, 8), jnp.float32),
        ],
    )(pre1, diff, wr, W_e2, b_e2, watt, batt, W_c1, b_c1, wc2)


def _make_scatter(E, N, D):
    C = 64
    NCHUNK = E // C
    J = pl.cdiv(NCHUNK, NW)
    NB = 2
    TT = -(-J // NB)
    mesh = plsc.VectorSubcoreMesh(core_axis_name="c", subcore_axis_name="s")

    @functools.partial(
        pl.kernel,
        mesh=mesh,
        out_type=[
            jax.ShapeDtypeStruct((NC, N, D), jnp.float32),
            jax.ShapeDtypeStruct((NC, N, 8), jnp.float32),
        ],
        scratch_types=(
            [pltpu.VMEM((2, C), jnp.int32) for _ in range(NB)]
            + [pltpu.VMEM((C, D), jnp.float32) for _ in range(NB)]
            + [pltpu.VMEM((C, 8), jnp.float32) for _ in range(NB)]
            + [pltpu.VMEM_SHARED((N, D), jnp.float32)]
            + [pltpu.VMEM_SHARED((N, 8), jnp.float32)]
            + [pltpu.SemaphoreType.DMA] * (5 * NB)
        ),
        compiler_params=pltpu.CompilerParams(use_tc_tiling_on_sc=False),
    )
    def scatter_kernel(ef_hbm, trans_hbm, rc2_hbm, z8_hbm,
                       agg_hbm, cagg_hbm, *sc):
        idx = sc[0:NB]
        F = sc[NB : 2 * NB]
        T = sc[2 * NB : 3 * NB]
        acc_sh = sc[3 * NB]
        cacc_sh = sc[3 * NB + 1]
        sems = sc[3 * NB + 2 :]
        semr = [sems[3 * b : 3 * b + 3] for b in range(NB)]
        sema = [sems[3 * NB + 2 * b : 3 * NB + 2 * b + 2] for b in range(NB)]

        core = lax.axis_index("c")
        sid = lax.axis_index("s")
        w = sid * NC + core

        def zf(i, carry):
            for k in range(D // 16):
                F[0][i, pl.ds(k * 16, 16)] = jnp.zeros((16,), jnp.float32)
            return carry

        lax.fori_loop(0, C, zf, 0)
        ra = 8 * ((N + 8 * NS - 1) // (8 * NS))
        base = sid * ra
        last = N - (NS - 1) * ra

        def row_chunks(total):
            return [C] * (total // C) + ([total % C] if total % C else [])

        for tail, sizes in ((False, row_chunks(ra)), (True, row_chunks(last))):
            @pl.when((sid == NS - 1) if tail else (sid < NS - 1))
            def _():
                off = 0
                for sz in sizes:
                    pltpu.sync_copy(F[0].at[pl.ds(0, sz)],
                                    acc_sh.at[pl.ds(base + off, sz)])
                    off += sz

        _dist_rows(z8_hbm, cacc_sh, sid, N)
        plsc.subcore_barrier()

        def cid(k):
            return jnp.minimum(w + k * NW, NCHUNK - 1)

        def real(k):
            return w + k * NW < NCHUNK

        def read_start(b, ch):
            pltpu.async_copy(rc2_hbm.at[ch], idx[b], semr[b][0])
            pltpu.async_copy(ef_hbm.at[pl.ds(ch * C, C)], F[b], semr[b][1])
            pltpu.async_copy(trans_hbm.at[pl.ds(ch * C, C)], T[b], semr[b][2])

        def read_wait(b):
            pltpu.make_async_copy(rc2_hbm.at[0], idx[b], semr[b][0]).wait()
            pltpu.make_async_copy(ef_hbm.at[pl.ds(0, C)], F[b], semr[b][1]).wait()
            pltpu.make_async_copy(trans_hbm.at[pl.ds(0, C)], T[b], semr[b][2]).wait()

        def adds_start(b):
            pltpu.async_copy(F[b], acc_sh.at[idx[b].at[0]], sema[b][0], add=True)
            pltpu.async_copy(T[b], cacc_sh.at[idx[b].at[0]], sema[b][1], add=True)

        def adds_wait(b):
            pltpu.make_async_copy(F[b], acc_sh.at[idx[b].at[0]], sema[b][0]).wait()
            pltpu.make_async_copy(T[b], cacc_sh.at[idx[b].at[0]], sema[b][1]).wait()

        for b in range(NB):
            read_start(b, cid(b))

        def loop_body(j, carry):
            for b in range(NB):
                read_wait(b)

                @pl.when(real(j * NB + b))
                def _():
                    adds_start(b)

            for b in range(NB):

                @pl.when(real(j * NB + b))
                def _():
                    adds_wait(b)

                read_start(b, cid(j * NB + b + NB))
            return carry

        lax.fori_loop(0, TT, loop_body, 0)
        for b in range(NB):
            read_wait(b)
        plsc.subcore_barrier()

        for tail, sizes in ((False, row_chunks(ra)), (True, row_chunks(last))):
            @pl.when((sid == NS - 1) if tail else (sid < NS - 1))
            def _():
                off = 0
                for sz in sizes:
                    pltpu.sync_copy(acc_sh.at[pl.ds(base + off, sz)],
                                    F[0].at[pl.ds(0, sz)])
                    pltpu.sync_copy(F[0].at[pl.ds(0, sz)],
                                    agg_hbm.at[core, pl.ds(base + off, sz)])
                    pltpu.sync_copy(cacc_sh.at[pl.ds(base + off, sz)],
                                    T[0].at[pl.ds(0, sz)])
                    pltpu.sync_copy(T[0].at[pl.ds(0, sz)],
                                    cagg_hbm.at[core, pl.ds(base + off, sz)])
                    off += sz

    return scatter_kernel


def _make_node_body(npart):
    def _node_body(
        h_ref, agg_ref, cagg_ref, crd_ref, wna_ref, wnb_ref, bn1_ref, wn2_ref,
        bn2_ref, hout_ref, cout_ref,
    ):
        h = h_ref[...]
        agg = agg_ref[0]
        cagg = cagg_ref[0]
        for q in range(1, npart):
            agg = agg + agg_ref[q]
            cagg = cagg + cagg_ref[q]
        x = jax.nn.silu(
            jnp.dot(h, wna_ref[...], preferred_element_type=jnp.float32)
            + jnp.dot(agg, wnb_ref[...], preferred_element_type=jnp.float32)
            + bn1_ref[...]
        )
        hout_ref[...] = (
            jnp.dot(x, wn2_ref[...], preferred_element_type=jnp.float32)
            + bn2_ref[...] + h
        )
        cout_ref[...] = crd_ref[...] + cagg

    return _node_body


def _node_mlp(h, agg2, cagg2, crd8, W_n1a, W_n1b, b_n1, W_n2, b_n2, bn=256):
    n, d = h.shape
    npart = agg2.shape[0]
    grid = (pl.cdiv(n, bn),)
    return pl.pallas_call(
        _make_node_body(npart),
        grid=grid,
        in_specs=[
            pl.BlockSpec((bn, d), lambda i: (i, 0)),
            pl.BlockSpec((npart, bn, d), lambda i: (0, i, 0)),
            pl.BlockSpec((npart, bn, 8), lambda i: (0, i, 0)),
            pl.BlockSpec((bn, 8), lambda i: (i, 0)),
            pl.BlockSpec((d, d), lambda i: (0, 0)),
            pl.BlockSpec((d, d), lambda i: (0, 0)),
            pl.BlockSpec((1, d), lambda i: (0, 0)),
            pl.BlockSpec((d, d), lambda i: (0, 0)),
            pl.BlockSpec((1, d), lambda i: (0, 0)),
        ],
        out_specs=[
            pl.BlockSpec((bn, d), lambda i: (i, 0)),
            pl.BlockSpec((bn, 8), lambda i: (i, 0)),
        ],
        out_shape=[
            jax.ShapeDtypeStruct((n, d), jnp.float32),
            jax.ShapeDtypeStruct((n, 8), jnp.float32),
        ],
    )(h, agg2, cagg2, crd8, W_n1a, W_n1b, b_n1, W_n2, b_n2)


def kernel(
    h, edge_index, coord,
    W_e1, b_e1, W_e2, b_e2, W_att, b_att,
    W_n1, b_n1, W_n2, b_n2, W_c1, b_c1, W_c2,
):
    N, D = h.shape
    E = edge_index.shape[1]
    assert E % 512 == 0 and D % 16 == 0 and N % NS == 0

    rc64 = edge_index.reshape(2, E // 64, 64).transpose(1, 0, 2)
    crd16 = jnp.pad(coord, ((0, 0), (0, 16 - coord.shape[1])))

    hs, hd = _precompute(h, W_e1[:D], W_e1[D : 2 * D], b_e1.reshape(1, D))

    K = 2
    ES = E // K
    RS = ES // 64
    gather_f = _make_gather(N, ES, D)
    scatter_f = _make_scatter(ES, N, D)
    wr = W_e1[2 * D].reshape(1, D)
    z8 = jnp.zeros((N, 8), jnp.float32)
    aggs, caggs = [], []
    for k in range(K):
        rck = rc64[k * RS : (k + 1) * RS]
        pre1, diff = gather_f(hs, hd, crd16, rck)
        ef, trans = _edge_mlp(
            pre1, diff, wr, W_e2, b_e2.reshape(1, -1),
            W_att.reshape(1, D), b_att.reshape(1, 1),
            W_c1, b_c1.reshape(1, -1), W_c2.reshape(1, D),
        )
        a2, c2 = scatter_f(ef, trans, rck, z8)
        aggs.append(a2)
        caggs.append(c2)
    agg2 = jnp.concatenate(aggs, axis=0)
    cagg2 = jnp.concatenate(caggs, axis=0)

    h_out, c8 = _node_mlp(
        h, agg2, cagg2, crd16[:, :8],
        W_n1[:D], W_n1[D:], b_n1.reshape(1, -1), W_n2, b_n2.reshape(1, -1),
    )
    return (h_out, c8[:, : coord.shape[1]])


# speedup vs baseline: 4.1231x; 1.0219x over previous
import functools

import jax
import jax.numpy as jnp
from jax import lax
from jax.experimental import pallas as pl
from jax.experimental.pallas import tpu as pltpu
from jax.experimental.pallas import tpu_sc as plsc

NC = 2
NS = 16
NW = NC * NS


def _pre_body(h_ref, wa_ref, wb_ref, be1_ref, hs_ref, hd_ref):
    h = h_ref[...]
    hs_ref[...] = jnp.dot(h, wa_ref[...], preferred_element_type=jnp.float32)
    hd_ref[...] = (
        jnp.dot(h, wb_ref[...], preferred_element_type=jnp.float32) + be1_ref[...]
    )


def _precompute(h, W_e1a, W_e1b, b_e1, bn=256):
    n, d = h.shape
    grid = (pl.cdiv(n, bn),)
    return pl.pallas_call(
        _pre_body,
        grid=grid,
        in_specs=[
            pl.BlockSpec((bn, d), lambda i: (i, 0)),
            pl.BlockSpec((d, d), lambda i: (0, 0)),
            pl.BlockSpec((d, d), lambda i: (0, 0)),
            pl.BlockSpec((1, d), lambda i: (0, 0)),
        ],
        out_specs=[
            pl.BlockSpec((bn, d), lambda i: (i, 0)),
            pl.BlockSpec((bn, d), lambda i: (i, 0)),
        ],
        out_shape=[
            jax.ShapeDtypeStruct((n, d), jnp.float32),
            jax.ShapeDtypeStruct((n, d), jnp.float32),
        ],
    )(h, W_e1a, W_e1b, b_e1)


def _dist_rows(src, dst, sid, n):
    ra = 8 * ((n + 8 * NS - 1) // (8 * NS))
    last = n - (NS - 1) * ra
    assert last > 0 and last % 8 == 0 and ra % 8 == 0

    @pl.when(sid < NS - 1)
    def _():
        pltpu.sync_copy(src.at[pl.ds(sid * ra, ra)], dst.at[pl.ds(sid * ra, ra)])

    @pl.when(sid == NS - 1)
    def _():
        pltpu.sync_copy(
            src.at[pl.ds((NS - 1) * ra, last)], dst.at[pl.ds((NS - 1) * ra, last)]
        )


def _make_gather(N, E, D):
    C = 128
    NCHUNK = E // C
    J = pl.cdiv(NCHUNK, NW)
    NB = 2
    TT = -(-J // NB)
    mesh = plsc.VectorSubcoreMesh(core_axis_name="c", subcore_axis_name="s")

    @functools.partial(
        pl.kernel,
        mesh=mesh,
        out_type=[
            jax.ShapeDtypeStruct((E, D), jnp.float32),
            jax.ShapeDtypeStruct((E, 16), jnp.float32),
        ],
        scratch_types=(
            [pltpu.VMEM((2, C), jnp.int32) for _ in range(NB)]
            + [pltpu.VMEM((C, D), jnp.float32) for _ in range(NB)]
            + [pltpu.VMEM((C, D), jnp.float32) for _ in range(NB)]
            + [pltpu.VMEM((C, 16), jnp.float32) for _ in range(NB)]
            + [pltpu.VMEM((C, 16), jnp.float32) for _ in range(NB)]
            + [pltpu.VMEM_SHARED((N, 16), jnp.float32)]
            + [pltpu.SemaphoreType.DMA] * (7 * NB)
        ),
        compiler_params=pltpu.CompilerParams(use_tc_tiling_on_sc=False),
    )
    def gather_kernel(hs_hbm, hd_hbm, crd_hbm, rc2_hbm, pre1_hbm, diff_hbm, *sc):
        idx = sc[0:NB]
        A = sc[NB : 2 * NB]
        B = sc[2 * NB : 3 * NB]
        R = sc[3 * NB : 4 * NB]
        Cc = sc[4 * NB : 5 * NB]
        crd_sh = sc[5 * NB]
        sems = sc[5 * NB + 1 :]
        semi = sems[0:NB]
        semg = [sems[NB + 4 * b : NB + 4 * b + 4] for b in range(NB)]
        semo = [sems[5 * NB + 2 * b : 5 * NB + 2 * b + 2] for b in range(NB)]

        core = lax.axis_index("c")
        sid = lax.axis_index("s")
        w = sid * NC + core

        _dist_rows(crd_hbm, crd_sh, sid, N)
        plsc.subcore_barrier()

        def cid(k):
            return jnp.minimum(w + k * NW, NCHUNK - 1)

        def idx_start(b, ch):
            pltpu.async_copy(rc2_hbm.at[ch], idx[b], semi[b])

        def idx_wait(b):
            pltpu.make_async_copy(rc2_hbm.at[0], idx[b], semi[b]).wait()

        def gather_start(b):
            pltpu.async_copy(hs_hbm.at[idx[b].at[0]], A[b], semg[b][0])
            pltpu.async_copy(hd_hbm.at[idx[b].at[1]], B[b], semg[b][1])
            pltpu.async_copy(crd_sh.at[idx[b].at[0]], R[b], semg[b][2])
            pltpu.async_copy(crd_sh.at[idx[b].at[1]], Cc[b], semg[b][3])

        def gather_wait(b):
            pltpu.make_async_copy(hs_hbm.at[idx[b].at[0]], A[b], semg[b][0]).wait()
            pltpu.make_async_copy(hd_hbm.at[idx[b].at[1]], B[b], semg[b][1]).wait()
            pltpu.make_async_copy(crd_hbm.at[pl.ds(0, C)], R[b], semg[b][2]).wait()
            pltpu.make_async_copy(crd_hbm.at[pl.ds(0, C)], Cc[b], semg[b][3]).wait()

        def compute(b):
            def edge_body(e, carry):
                for k in range(D // 16):
                    s = pl.ds(k * 16, 16)
                    A[b][e, s] = A[b][e, s] + B[b][e, s]
                R[b][e, :] = R[b][e, :] - Cc[b][e, :]
                return carry

            lax.fori_loop(0, C, edge_body, 0)

        def out_start(b, ch):
            pltpu.async_copy(A[b], pre1_hbm.at[pl.ds(ch * C, C)], semo[b][0])
            pltpu.async_copy(R[b], diff_hbm.at[pl.ds(ch * C, C)], semo[b][1])

        def out_wait(b):
            pltpu.make_async_copy(A[b], pre1_hbm.at[pl.ds(0, C)], semo[b][0]).wait()
            pltpu.make_async_copy(R[b], diff_hbm.at[pl.ds(0, C)], semo[b][1]).wait()

        def step(j, b, first):
            bn = (b + 1) % NB
            if not (first and b < NB - 1):
                out_wait(bn)
            idx_wait(bn)
            gather_start(bn)
            gather_wait(b)
            compute(b)
            out_start(b, cid(j * NB + b))
            idx_start(b, cid(j * NB + b + NB))

        for b in range(NB):
            idx_start(b, cid(b))
        idx_wait(0)
        gather_start(0)

        for b in range(NB):
            step(0, b, True)

        def loop_body(j, carry):
            for b in range(NB):
                step(j, b, False)
            return carry

        lax.fori_loop(1, TT, loop_body, 0)

        for b in range(1, NB):
            out_wait(b)
            idx_wait(b)
        gather_wait(0)

    return gather_kernel


def _edge_body(
    pre1_ref, diff_ref, wr_ref, we2_ref, be2_ref, watt_ref, batt_ref,
    wc1_ref, bc1_ref, wc2_ref, ef_ref, trans_ref,
):
    d = diff_ref[...]
    radial = jnp.sum(d * d, axis=1, keepdims=True)
    m1 = jax.nn.silu(pre1_ref[...] + radial * wr_ref[...])
    m2 = jax.nn.silu(
        jnp.dot(m1, we2_ref[...], preferred_element_type=jnp.float32) + be2_ref[...]
    )
    att_logit = jnp.sum(m2 * watt_ref[...], axis=1, keepdims=True) + batt_ref[0, 0]
    ef = m2 * jax.nn.sigmoid(att_logit)
    cm = jax.nn.silu(
        jnp.dot(ef, wc1_ref[...], preferred_element_type=jnp.float32) + bc1_ref[...]
    )
    c = jnp.sum(cm * wc2_ref[...], axis=1, keepdims=True)
    ef_ref[...] = ef
    trans_ref[...] = (d * c)[:, :8]


def _edge_mlp(pre1, diff, wr, W_e2, b_e2, watt, batt, W_c1, b_c1, wc2, be=512):
    E, D = pre1.shape
    grid = (E // be,)
    return pl.pallas_call(
        _edge_body,
        grid=grid,
        in_specs=[
            pl.BlockSpec((be, D), lambda i: (i, 0)),
            pl.BlockSpec((be, 16), lambda i: (i, 0)),
            pl.BlockSpec((1, D), lambda i: (0, 0)),
            pl.BlockSpec((D, D), lambda i: (0, 0)),
            pl.BlockSpec((1, D), lambda i: (0, 0)),
            pl.BlockSpec((1, D), lambda i: (0, 0)),
            pl.BlockSpec((1, 1), lambda i: (0, 0)),
            pl.BlockSpec((D, D), lambda i: (0, 0)),
            pl.BlockSpec((1, D), lambda i: (0, 0)),
            pl.BlockSpec((1, D), lambda i: (0, 0)),
        ],
        out_specs=[
            pl.BlockSpec((be, D), lambda i: (i, 0)),
            pl.BlockSpec((be, 8), lambda i: (i, 0)),
        ],
        out_shape=[
            jax.ShapeDtypeStruct((E, D), jnp.float32),
            jax.ShapeDtypeStruct((E, 8), jnp.float32),
        ],
    )(pre1, diff, wr, W_e2, b_e2, watt, batt, W_c1, b_c1, wc2)


def _make_scatter(E, N, D):
    C = 128
    NCHUNK = E // C
    J = pl.cdiv(NCHUNK, NW)
    NB = 2
    TT = -(-J // NB)
    mesh = plsc.VectorSubcoreMesh(core_axis_name="c", subcore_axis_name="s")

    @functools.partial(
        pl.kernel,
        mesh=mesh,
        out_type=[
            jax.ShapeDtypeStruct((NC, N, D), jnp.float32),
            jax.ShapeDtypeStruct((NC, N, 8), jnp.float32),
        ],
        scratch_types=(
            [pltpu.VMEM((2, C), jnp.int32) for _ in range(NB)]
            + [pltpu.VMEM((C, D), jnp.float32) for _ in range(NB)]
            + [pltpu.VMEM((C, 8), jnp.float32) for _ in range(NB)]
            + [pltpu.VMEM_SHARED((N, D), jnp.float32)]
            + [pltpu.VMEM_SHARED((N, 8), jnp.float32)]
            + [pltpu.SemaphoreType.DMA] * (5 * NB)
        ),
        compiler_params=pltpu.CompilerParams(use_tc_tiling_on_sc=False),
    )
    def scatter_kernel(ef_hbm, trans_hbm, rc2_hbm, z8_hbm,
                       agg_hbm, cagg_hbm, *sc):
        idx = sc[0:NB]
        F = sc[NB : 2 * NB]
        T = sc[2 * NB : 3 * NB]
        acc_sh = sc[3 * NB]
        cacc_sh = sc[3 * NB + 1]
        sems = sc[3 * NB + 2 :]
        semr = [sems[3 * b : 3 * b + 3] for b in range(NB)]
        sema = [sems[3 * NB + 2 * b : 3 * NB + 2 * b + 2] for b in range(NB)]

        core = lax.axis_index("c")
        sid = lax.axis_index("s")
        w = sid * NC + core

        def zf(i, carry):
            for k in range(D // 16):
                F[0][i, pl.ds(k * 16, 16)] = jnp.zeros((16,), jnp.float32)
            return carry

        lax.fori_loop(0, C, zf, 0)
        ra = 8 * ((N + 8 * NS - 1) // (8 * NS))
        base = sid * ra
        last = N - (NS - 1) * ra

        def row_chunks(total):
            return [C] * (total // C) + ([total % C] if total % C else [])

        for tail, sizes in ((False, row_chunks(ra)), (True, row_chunks(last))):
            @pl.when((sid == NS - 1) if tail else (sid < NS - 1))
            def _():
                off = 0
                for sz in sizes:
                    pltpu.sync_copy(F[0].at[pl.ds(0, sz)],
                                    acc_sh.at[pl.ds(base + off, sz)])
                    off += sz

        _dist_rows(z8_hbm, cacc_sh, sid, N)
        plsc.subcore_barrier()

        def cid(k):
            return jnp.minimum(w + k * NW, NCHUNK - 1)

        def real(k):
            return w + k * NW < NCHUNK

        def read_start(b, ch):
            pltpu.async_copy(rc2_hbm.at[ch], idx[b], semr[b][0])
            pltpu.async_copy(ef_hbm.at[pl.ds(ch * C, C)], F[b], semr[b][1])
            pltpu.async_copy(trans_hbm.at[pl.ds(ch * C, C)], T[b], semr[b][2])

        def read_wait(b):
            pltpu.make_async_copy(rc2_hbm.at[0], idx[b], semr[b][0]).wait()
            pltpu.make_async_copy(ef_hbm.at[pl.ds(0, C)], F[b], semr[b][1]).wait()
            pltpu.make_async_copy(trans_hbm.at[pl.ds(0, C)], T[b], semr[b][2]).wait()

        def adds_start(b):
            pltpu.async_copy(F[b], acc_sh.at[idx[b].at[0]], sema[b][0], add=True)
            pltpu.async_copy(T[b], cacc_sh.at[idx[b].at[0]], sema[b][1], add=True)

        def adds_wait(b):
            pltpu.make_async_copy(F[b], acc_sh.at[idx[b].at[0]], sema[b][0]).wait()
            pltpu.make_async_copy(T[b], cacc_sh.at[idx[b].at[0]], sema[b][1]).wait()

        for b in range(NB):
            read_start(b, cid(b))

        def loop_body(j, carry):
            for b in range(NB):
                read_wait(b)

                @pl.when(real(j * NB + b))
                def _():
                    adds_start(b)

            for b in range(NB):

                @pl.when(real(j * NB + b))
                def _():
                    adds_wait(b)

                read_start(b, cid(j * NB + b + NB))
            return carry

        lax.fori_loop(0, TT, loop_body, 0)
        for b in range(NB):
            read_wait(b)
        plsc.subcore_barrier()

        for tail, sizes in ((False, row_chunks(ra)), (True, row_chunks(last))):
            @pl.when((sid == NS - 1) if tail else (sid < NS - 1))
            def _():
                off = 0
                for sz in sizes:
                    pltpu.sync_copy(acc_sh.at[pl.ds(base + off, sz)],
                                    F[0].at[pl.ds(0, sz)])
                    pltpu.sync_copy(F[0].at[pl.ds(0, sz)],
                                    agg_hbm.at[core, pl.ds(base + off, sz)])
                    pltpu.sync_copy(cacc_sh.at[pl.ds(base + off, sz)],
                                    T[0].at[pl.ds(0, sz)])
                    pltpu.sync_copy(T[0].at[pl.ds(0, sz)],
                                    cagg_hbm.at[core, pl.ds(base + off, sz)])
                    off += sz

    return scatter_kernel


def _make_node_body(npart):
    def _node_body(
        h_ref, agg_ref, cagg_ref, crd_ref, wna_ref, wnb_ref, bn1_ref, wn2_ref,
        bn2_ref, hout_ref, cout_ref,
    ):
        h = h_ref[...]
        agg = agg_ref[0]
        cagg = cagg_ref[0]
        for q in range(1, npart):
            agg = agg + agg_ref[q]
            cagg = cagg + cagg_ref[q]
        x = jax.nn.silu(
            jnp.dot(h, wna_ref[...], preferred_element_type=jnp.float32)
            + jnp.dot(agg, wnb_ref[...], preferred_element_type=jnp.float32)
            + bn1_ref[...]
        )
        hout_ref[...] = (
            jnp.dot(x, wn2_ref[...], preferred_element_type=jnp.float32)
            + bn2_ref[...] + h
        )
        cout_ref[...] = crd_ref[...] + cagg

    return _node_body


def _node_mlp(h, agg2, cagg2, crd8, W_n1a, W_n1b, b_n1, W_n2, b_n2, bn=256):
    n, d = h.shape
    npart = agg2.shape[0]
    grid = (pl.cdiv(n, bn),)
    return pl.pallas_call(
        _make_node_body(npart),
        grid=grid,
        in_specs=[
            pl.BlockSpec((bn, d), lambda i: (i, 0)),
            pl.BlockSpec((npart, bn, d), lambda i: (0, i, 0)),
            pl.BlockSpec((npart, bn, 8), lambda i: (0, i, 0)),
            pl.BlockSpec((bn, 8), lambda i: (i, 0)),
            pl.BlockSpec((d, d), lambda i: (0, 0)),
            pl.BlockSpec((d, d), lambda i: (0, 0)),
            pl.BlockSpec((1, d), lambda i: (0, 0)),
            pl.BlockSpec((d, d), lambda i: (0, 0)),
            pl.BlockSpec((1, d), lambda i: (0, 0)),
        ],
        out_specs=[
            pl.BlockSpec((bn, d), lambda i: (i, 0)),
            pl.BlockSpec((bn, 8), lambda i: (i, 0)),
        ],
        out_shape=[
            jax.ShapeDtypeStruct((n, d), jnp.float32),
            jax.ShapeDtypeStruct((n, 8), jnp.float32),
        ],
    )(h, agg2, cagg2, crd8, W_n1a, W_n1b, b_n1, W_n2, b_n2)


def kernel(
    h, edge_index, coord,
    W_e1, b_e1, W_e2, b_e2, W_att, b_att,
    W_n1, b_n1, W_n2, b_n2, W_c1, b_c1, W_c2,
):
    N, D = h.shape
    E = edge_index.shape[1]
    assert E % 512 == 0 and D % 16 == 0 and N % NS == 0

    rc128 = edge_index.reshape(2, E // 128, 128).transpose(1, 0, 2)
    crd16 = jnp.pad(coord, ((0, 0), (0, 16 - coord.shape[1])))

    hs, hd = _precompute(h, W_e1[:D], W_e1[D : 2 * D], b_e1.reshape(1, D))

    K = 2
    ES = E // K
    RS = ES // 128
    gather_f = _make_gather(N, ES, D)
    scatter_f = _make_scatter(ES, N, D)
    wr = W_e1[2 * D].reshape(1, D)
    z8 = jnp.zeros((N, 8), jnp.float32)
    aggs, caggs = [], []
    for k in range(K):
        rck = rc128[k * RS : (k + 1) * RS]
        pre1, diff = gather_f(hs, hd, crd16, rck)
        ef, trans = _edge_mlp(
            pre1, diff, wr, W_e2, b_e2.reshape(1, -1),
            W_att.reshape(1, D), b_att.reshape(1, 1),
            W_c1, b_c1.reshape(1, -1), W_c2.reshape(1, D),
        )
        a2, c2 = scatter_f(ef, trans, rck, z8)
        aggs.append(a2)
        caggs.append(c2)
    agg2 = jnp.concatenate(aggs, axis=0)
    cagg2 = jnp.concatenate(caggs, axis=0)

    h_out, c8 = _node_mlp(
        h, agg2, cagg2, crd16[:, :8],
        W_n1[:D], W_n1[D:], b_n1.reshape(1, -1), W_n2, b_n2.reshape(1, -1),
    )
    return (h_out, c8[:, : coord.shape[1]])
